# baseline (speedup 1.0000x reference)
import jax
import jax.numpy as jnp
from jax import lax
from jax.experimental import pallas as pl
from jax.experimental.pallas import tpu as pltpu
from jax.experimental.pallas import tpu_sc as plsc

D_MODEL = 64
SEQ = 200
NC, NS = 2, 16
NW = NC * NS
CHR = 128
LANES = 16
VPR = D_MODEL // LANES


def _emb_body(x_hbm, table_hbm, pos_hbm, out_hbm,
              idx_v, pos_v, gbuf, obuf,
              gsem0, gsem1, osem0, osem1):
    nch = x_hbm.shape[0] // NW
    rows_w = nch * CHR
    wid = lax.axis_index("s") * NC + lax.axis_index("c")

    pltpu.sync_copy(x_hbm.at[pl.ds(wid * nch, nch)], idx_v)
    pltpu.sync_copy(pos_hbm, pos_v.at[pl.ds(0, SEQ)])
    pltpu.sync_copy(pos_hbm, pos_v.at[pl.ds(SEQ, SEQ)])

    gsems = (gsem0, gsem1)
    osems = (osem0, osem1)

    def gather_copy(c, buf):
        return pltpu.make_async_copy(
            table_hbm.at[idx_v.at[c]], gbuf.at[buf], gsems[buf])

    def out_copy(c, buf):
        row0 = wid * rows_w + c * CHR
        return pltpu.make_async_copy(
            obuf.at[buf],
            out_hbm.at[pl.ds(row0, CHR), pl.ds(0, D_MODEL)],
            osems[buf])

    gather_copy(0, 0).start()
    gather_copy(1, 1).start()

    def chunk(t, b):
        c = 2 * t + b
        gather_copy(c, b).wait()
        @pl.when(t > 0)
        def _():
            out_copy(c - 2, b).wait()

        s0 = lax.rem(c * CHR, SEQ)

        def add_rows(r, _):
            for u in range(2):
                rr = 2 * r + u
                for j in range(VPR):
                    sl = pl.ds(j * LANES, LANES)
                    obuf[b, rr, sl] = gbuf[b, rr, sl] + pos_v[s0 + rr, sl]
            return 0

        lax.fori_loop(0, CHR // 2, add_rows, 0)

        @pl.when(c + 2 < nch)
        def _():
            gather_copy(c + 2, b).start()

        out_copy(c, b).start()

    def step(t, _):
        chunk(t, 0)
        chunk(t, 1)
        return 0

    lax.fori_loop(0, nch // 2, step, 0)

    for b in range(2):
        out_copy(nch - 2 + b, b).wait()


def kernel(x, token_table, pos_table):
    B, S = x.shape
    total = B * S
    x_lin = x.astype(jnp.int32).reshape(total // 128, 128)

    mesh = plsc.VectorSubcoreMesh(core_axis_name="c", subcore_axis_name="s")
    out = pl.kernel(
        _emb_body,
        out_type=jax.ShapeDtypeStruct((total, 128), jnp.float32),
        mesh=mesh,
        compiler_params=pltpu.CompilerParams(use_tc_tiling_on_sc=False),
        scratch_types=[
            pltpu.VMEM((total // (NW * CHR), CHR), jnp.int32),
            pltpu.VMEM((2 * SEQ, D_MODEL), jnp.float32),
            pltpu.VMEM((2, CHR, D_MODEL), jnp.float32),
            pltpu.VMEM((2, CHR, D_MODEL), jnp.float32),
            pltpu.SemaphoreType.DMA,
            pltpu.SemaphoreType.DMA,
            pltpu.SemaphoreType.DMA,
            pltpu.SemaphoreType.DMA,
        ],
    )(x_lin, token_table, pos_table)
    return out[:, :D_MODEL].reshape(B, S, D_MODEL)
